# baseline (speedup 1.0000x reference)
import jax
import jax.numpy as jnp
from jax.experimental import pallas as pl
from jax.experimental.pallas import tpu as pltpu


def _round_up(x: int, m: int) -> int:
    return ((x + m - 1) // m) * m


_TM = 1024


def _fused_kernel(x_ref, w_ref, b_ref, o_ref, w_f32, w_bf, w_sem):
    @pl.when(pl.program_id(1) == 0)
    def _cast_weights():
        copy = pltpu.make_async_copy(w_ref, w_f32, w_sem)
        copy.start()
        copy.wait()
        w_bf[...] = w_f32[...].astype(jnp.bfloat16)

    xb = x_ref[...].astype(jnp.bfloat16)
    l1 = (
        jnp.dot(xb, w_bf[...], preferred_element_type=jnp.float32)
        + b_ref[...]
    )
    o_ref[...] = l1 * ((jnp.clip(l1, 0.0, 6.0) + 3.0) * (1.0 / 6.0))


@jax.jit
def kernel(x, w_p, b_p):
    B, in_f = x.shape
    INp = w_p.shape[0]
    OUTp = w_p.shape[1]

    tm = min(_TM, _round_up(B, 8))
    n_tiles = _round_up(B, tm) // tm
    n_cores = 2 if n_tiles % 2 == 0 else 1
    steps = n_tiles // n_cores
    Bp = n_tiles * tm
    if (Bp != B) or (INp != in_f):
        x = jnp.pad(x, ((0, Bp - B), (0, INp - in_f)))

    return pl.pallas_call(
        _fused_kernel,
        out_shape=jax.ShapeDtypeStruct((Bp, OUTp), jnp.float32),
        grid=(n_cores, steps),
        in_specs=[
            pl.BlockSpec((tm, INp), lambda i, j, c=n_cores: (j * c + i, 0)),
            pl.BlockSpec(memory_space=pl.ANY),
            pl.BlockSpec((1, OUTp), lambda i, j: (0, 0)),
        ],
        out_specs=pl.BlockSpec((tm, OUTp), lambda i, j, c=n_cores: (j * c + i, 0)),
        scratch_shapes=[
            pltpu.VMEM((INp, OUTp), jnp.float32),
            pltpu.VMEM((INp, OUTp), jnp.bfloat16),
            pltpu.SemaphoreType.DMA,
        ],
        compiler_params=pltpu.CompilerParams(
            dimension_semantics=("parallel", "arbitrary"),
        ),
    )(x, w_p, b_p)


# speedup vs baseline: 1.1789x; 1.1789x over previous
import jax
import jax.numpy as jnp
from jax.experimental import pallas as pl
from jax.experimental.pallas import tpu as pltpu


def _round_up(x: int, m: int) -> int:
    return ((x + m - 1) // m) * m


_TM = 1024


def _fused_kernel(x_ref, w_ref, b_ref, o_ref, w_bf):
    @pl.when(pl.program_id(1) == 0)
    def _cast_weights():
        w_bf[...] = w_ref[...].astype(jnp.bfloat16)

    xb = x_ref[...].astype(jnp.bfloat16)
    l1 = (
        jnp.dot(xb, w_bf[...], preferred_element_type=jnp.float32)
        + b_ref[...]
    )
    o_ref[...] = l1 * ((jnp.clip(l1, 0.0, 6.0) + 3.0) * (1.0 / 6.0))


@jax.jit
def kernel(x, w_p, b_p):
    B, in_f = x.shape
    INp = w_p.shape[0]
    OUTp = w_p.shape[1]

    tm = min(_TM, _round_up(B, 8))
    n_tiles = _round_up(B, tm) // tm
    n_cores = 2 if n_tiles % 2 == 0 else 1
    steps = n_tiles // n_cores
    Bp = n_tiles * tm
    if (Bp != B) or (INp != in_f):
        x = jnp.pad(x, ((0, Bp - B), (0, INp - in_f)))

    return pl.pallas_call(
        _fused_kernel,
        out_shape=jax.ShapeDtypeStruct((Bp, OUTp), jnp.float32),
        grid=(n_cores, steps),
        in_specs=[
            pl.BlockSpec((tm, INp), lambda i, j, c=n_cores: (j * c + i, 0)),
            pl.BlockSpec((INp, OUTp), lambda i, j: (0, 0)),
            pl.BlockSpec((1, OUTp), lambda i, j: (0, 0)),
        ],
        out_specs=pl.BlockSpec((tm, OUTp), lambda i, j, c=n_cores: (j * c + i, 0)),
        scratch_shapes=[pltpu.VMEM((INp, OUTp), jnp.bfloat16)],
        compiler_params=pltpu.CompilerParams(
            dimension_semantics=("parallel", "arbitrary"),
        ),
    )(x, w_p, b_p)
